# Initial kernel scaffold; baseline (speedup 1.0000x reference)
import jax
import jax.numpy as jnp
from jax.experimental import pallas as pl


def kernel(embedded_words, action_vectors, W, b):
    raise NotImplementedError("write your pallas kernel here")


# speedup vs baseline: 5.3875x; 5.3875x over previous
import functools

import jax
import jax.numpy as jnp
from jax import lax
from jax.experimental import pallas as pl
from jax.experimental.pallas import tpu as pltpu
from jax.experimental.pallas import tpu_sc as plsc

B = 16384
D = 64
P = 11
A = 4
LANES = 16

NC = 2
NS = 16
NW = NC * NS
ROWS_W = B // NW
CHUNK = 256
G2 = 32
EPS = 1e-8

GROUPS = ((0, 1, 2, 3), (4, 5, 6, 7, 8), (9,), (10,))


def _rsqrt(x):
    i = lax.bitcast_convert_type(x, jnp.int32)
    i = jnp.int32(0x5F3759DF) - (i >> 1)
    y = lax.bitcast_convert_type(i, jnp.float32)
    for _ in range(3):
        y = y * (1.5 - 0.5 * x * y * y)
    return y


def _maxtree(vals):
    vals = list(vals)
    while len(vals) > 1:
        vals = [jnp.maximum(vals[i], vals[i + 1]) for i in range(0, len(vals) - 1, 2)] + (
            [vals[-1]] if len(vals) % 2 else [])
    return vals[0]


def _sc_body(e_hbm, av_hbm, wb_hbm, out_hbm, e_v, out_v, av_v, wb_v):
    wid = lax.axis_index("s") * NC + lax.axis_index("c")
    base = wid * ROWS_W

    pltpu.sync_copy(av_hbm, av_v)
    pltpu.sync_copy(wb_hbm, wb_v)

    inv_nb = []
    for p in range(P):
        nb2 = jnp.zeros((LANES,), jnp.float32)
        for d in range(D):
            s = av_v[d * P + p, :]
            nb2 = nb2 + s * s
        inv_nb.append(_rsqrt(jnp.maximum(nb2, EPS * EPS)))

    lane = lax.iota(jnp.int32, LANES)

    def finish(acc, n2, rows):
        ca = _rsqrt(jnp.maximum(n2, EPS * EPS))
        sims = [acc[p] * inv_nb[p] for p in range(P)]
        pooled = [_maxtree([sims[p] for p in g]) * ca for g in GROUPS]
        logits = []
        for j in range(A):
            lj = pooled[0] * wb_v[4 * j + 0, :]
            for k in range(1, A):
                lj = lj + pooled[k] * wb_v[4 * j + k, :]
            logits.append(lj + wb_v[A * A + j, :])
        m = _maxtree(logits)
        es = [jnp.exp(l - m) for l in logits]
        tot = (es[0] + es[1]) + (es[2] + es[3])
        rs = _rsqrt(tot)
        inv = rs * rs
        for j in range(A):
            plsc.store_scatter(out_v, [rows, jnp.full((LANES,), j, jnp.int32)],
                               es[j] * inv)

    def chunk_body(c, carry):
        pltpu.sync_copy(e_hbm.at[pl.ds(base + c * CHUNK, CHUNK)], e_v)

        def blk_body(k, carry2):
            r0 = k * G2
            rows_a = r0 + lane
            rows_b = rows_a + LANES
            acc_a = [jnp.zeros((LANES,), jnp.float32) for _ in range(P)]
            acc_b = [jnp.zeros((LANES,), jnp.float32) for _ in range(P)]
            n2a = jnp.zeros((LANES,), jnp.float32)
            n2b = jnp.zeros((LANES,), jnp.float32)
            for d in range(D):
                dv = jnp.full((LANES,), d, jnp.int32)
                va = plsc.load_gather(e_v, [rows_a, dv])
                vb = plsc.load_gather(e_v, [rows_b, dv])
                for p in range(P):
                    s = av_v[d * P + p, :]
                    acc_a[p] = acc_a[p] + va * s
                    acc_b[p] = acc_b[p] + vb * s
                n2a = n2a + va * va
                n2b = n2b + vb * vb
            out_r0 = c * CHUNK + r0
            finish(acc_a, n2a, out_r0 + lane)
            finish(acc_b, n2b, out_r0 + LANES + lane)
            return carry2

        return lax.fori_loop(0, CHUNK // G2, blk_body, carry)

    lax.fori_loop(0, ROWS_W // CHUNK, chunk_body, 0)
    pltpu.sync_copy(out_v, out_hbm.at[pl.ds(base, ROWS_W)])


@functools.partial(
    pl.kernel,
    out_type=jax.ShapeDtypeStruct((B, A), jnp.float32),
    mesh=plsc.VectorSubcoreMesh(core_axis_name="c", subcore_axis_name="s"),
    compiler_params=pltpu.CompilerParams(needs_layout_passes=False,
                                         use_tc_tiling_on_sc=False),
    scratch_types=[
        pltpu.VMEM((CHUNK, D), jnp.float32),
        pltpu.VMEM((ROWS_W, A), jnp.float32),
        pltpu.VMEM((D * P, LANES), jnp.float32),
        pltpu.VMEM((A * A + A, LANES), jnp.float32),
    ],
)
def _sc_decoder(e_hbm, av_hbm, wb_hbm, out_hbm, e_v, out_v, av_v, wb_v):
    _sc_body(e_hbm, av_hbm, wb_hbm, out_hbm, e_v, out_v, av_v, wb_v)


def kernel(embedded_words, action_vectors, W, b):
    av = action_vectors.reshape(D * P).astype(jnp.float32)
    av_splat = jnp.broadcast_to(av[:, None], (D * P, LANES))
    wb = jnp.concatenate([W.reshape(A * A), b.reshape(A)]).astype(jnp.float32)
    wb_splat = jnp.broadcast_to(wb[:, None], (A * A + A, LANES))
    return _sc_decoder(embedded_words, av_splat, wb_splat)


# speedup vs baseline: 6.2315x; 1.1566x over previous
import functools

import numpy as np

import jax
import jax.numpy as jnp
from jax import lax
from jax.experimental import pallas as pl
from jax.experimental.pallas import tpu as pltpu
from jax.experimental.pallas import tpu_sc as plsc

B = 16384
D = 64
P = 11
A = 4
LANES = 16

NC = 2
NS = 16
NW = NC * NS
ROWS_W = B // NW
CHUNK = 256
G2 = 32
EPS = 1e-8

GROUPS = ((0, 1, 2, 3), (4, 5, 6, 7, 8), (9,), (10,))


def _rsqrt(x):
    i = lax.bitcast_convert_type(x, jnp.int32)
    i = jnp.int32(0x5F3759DF) - (i >> 1)
    y = lax.bitcast_convert_type(i, jnp.float32)
    for _ in range(3):
        y = y * (1.5 - 0.5 * x * y * y)
    return y


def _maxtree(vals):
    vals = list(vals)
    while len(vals) > 1:
        vals = [jnp.maximum(vals[i], vals[i + 1]) for i in range(0, len(vals) - 1, 2)] + (
            [vals[-1]] if len(vals) % 2 else [])
    return vals[0]


def _sc_body(e_hbm, av_hbm, wb_hbm, out_hbm, e_v, out_v, av_v, wb_v):
    wid = lax.axis_index("s") * NC + lax.axis_index("c")
    base = wid * ROWS_W

    pltpu.sync_copy(av_hbm, av_v)
    pltpu.sync_copy(wb_hbm, wb_v)

    inv_nb = []
    for p in range(P):
        nb2 = jnp.zeros((LANES,), jnp.float32)
        for d in range(D):
            s = av_v[d * P + p, :]
            nb2 = nb2 + s * s
        inv_nb.append(_rsqrt(jnp.maximum(nb2, EPS * EPS)))

    lane = lax.iota(jnp.int32, LANES)

    def finish(acc, n2, rows):
        ca = _rsqrt(jnp.maximum(n2, EPS * EPS))
        sims = [acc[p] * inv_nb[p] for p in range(P)]
        pooled = [_maxtree([sims[p] for p in g]) * ca for g in GROUPS]
        logits = []
        for j in range(A):
            lj = pooled[0] * wb_v[4 * j + 0, :]
            for k in range(1, A):
                lj = lj + pooled[k] * wb_v[4 * j + k, :]
            logits.append(lj + wb_v[A * A + j, :])
        m = _maxtree(logits)
        es = [jnp.exp(l - m) for l in logits]
        tot = (es[0] + es[1]) + (es[2] + es[3])
        rs = _rsqrt(tot)
        inv = rs * rs
        for j in range(A):
            plsc.store_scatter(out_v, [rows, jnp.full((LANES,), j, jnp.int32)],
                               es[j] * inv)

    def chunk_body(c, carry):
        pltpu.sync_copy(e_hbm.at[pl.ds(base + c * CHUNK, CHUNK)], e_v)

        def blk_body(k, carry2):
            r0 = k * G2
            rows_a = r0 + lane
            rows_b = rows_a + LANES
            acc_a = [jnp.zeros((LANES,), jnp.float32) for _ in range(P)]
            acc_b = [jnp.zeros((LANES,), jnp.float32) for _ in range(P)]
            n2a = jnp.zeros((LANES,), jnp.float32)
            n2b = jnp.zeros((LANES,), jnp.float32)
            for d in range(D):
                dv = (lane + d) & (D - 1)
                va = plsc.load_gather(e_v, [rows_a, dv])
                vb = plsc.load_gather(e_v, [rows_b, dv])
                for p in range(P):
                    s = av_v[d * P + p, :]
                    acc_a[p] = acc_a[p] + va * s
                    acc_b[p] = acc_b[p] + vb * s
                n2a = n2a + va * va
                n2b = n2b + vb * vb
            out_r0 = c * CHUNK + r0
            finish(acc_a, n2a, out_r0 + lane)
            finish(acc_b, n2b, out_r0 + LANES + lane)
            return carry2

        return lax.fori_loop(0, CHUNK // G2, blk_body, carry)

    lax.fori_loop(0, ROWS_W // CHUNK, chunk_body, 0)
    pltpu.sync_copy(out_v, out_hbm.at[pl.ds(base, ROWS_W)])


@functools.partial(
    pl.kernel,
    out_type=jax.ShapeDtypeStruct((B, A), jnp.float32),
    mesh=plsc.VectorSubcoreMesh(core_axis_name="c", subcore_axis_name="s"),
    compiler_params=pltpu.CompilerParams(needs_layout_passes=False,
                                         use_tc_tiling_on_sc=False),
    scratch_types=[
        pltpu.VMEM((CHUNK, D), jnp.float32),
        pltpu.VMEM((ROWS_W, A), jnp.float32),
        pltpu.VMEM((D * P, LANES), jnp.float32),
        pltpu.VMEM((A * A + A, LANES), jnp.float32),
    ],
)
def _sc_decoder(e_hbm, av_hbm, wb_hbm, out_hbm, e_v, out_v, av_v, wb_v):
    _sc_body(e_hbm, av_hbm, wb_hbm, out_hbm, e_v, out_v, av_v, wb_v)


def kernel(embedded_words, action_vectors, W, b):
    av2 = action_vectors.reshape(D, P).astype(jnp.float32)
    rot = (np.arange(D)[:, None] + np.arange(LANES)[None, :]) % D
    av_splat = jnp.transpose(av2[rot, :], (0, 2, 1)).reshape(D * P, LANES)
    wb = jnp.concatenate([W.reshape(A * A), b.reshape(A)]).astype(jnp.float32)
    wb_splat = jnp.broadcast_to(wb[:, None], (A * A + A, LANES))
    return _sc_decoder(embedded_words, av_splat, wb_splat)


# speedup vs baseline: 10.9862x; 1.7630x over previous
import functools

import jax
import jax.numpy as jnp
from jax import lax
from jax.experimental import pallas as pl
from jax.experimental.pallas import tpu as pltpu
from jax.experimental.pallas import tpu_sc as plsc

B = 16384
D = 64
P = 11
A = 4
LANES = 16

NC = 2
NS = 16
NW = NC * NS
ROWS_W = B // NW
G2 = 32
EPS = 1e-8

GROUPS = ((0, 1, 2, 3), (4, 5, 6, 7, 8), (9,), (10,))


def _rsqrt(x):
    i = lax.bitcast_convert_type(x, jnp.int32)
    i = jnp.int32(0x5F3759DF) - (i >> 1)
    y = lax.bitcast_convert_type(i, jnp.float32)
    for _ in range(3):
        y = y * (1.5 - 0.5 * x * y * y)
    return y


def _maxtree(vals):
    vals = list(vals)
    while len(vals) > 1:
        vals = [jnp.maximum(vals[i], vals[i + 1]) for i in range(0, len(vals) - 1, 2)] + (
            [vals[-1]] if len(vals) % 2 else [])
    return vals[0]


def _sc_body(et_hbm, av_hbm, wb_hbm, out_hbm, et_v, out_v, av_v, wb_v):
    wid = lax.axis_index("s") * NC + lax.axis_index("c")

    pltpu.sync_copy(av_hbm, av_v)
    pltpu.sync_copy(wb_hbm, wb_v)
    pltpu.sync_copy(et_hbm.at[:, pl.ds(wid * (ROWS_W // 128), ROWS_W // 128)], et_v)

    inv_nb = []
    for p in range(P):
        nb2 = jnp.zeros((LANES,), jnp.float32)
        for d in range(D):
            s = av_v[d * P + p, :]
            nb2 = nb2 + s * s
        inv_nb.append(_rsqrt(jnp.maximum(nb2, EPS * EPS)))

    def finish(acc, n2, lr0):
        ca = _rsqrt(jnp.maximum(n2, EPS * EPS))
        sims = [acc[p] * inv_nb[p] for p in range(P)]
        pooled = [_maxtree([sims[p] for p in g]) * ca for g in GROUPS]
        logits = []
        for j in range(A):
            lj = pooled[0] * wb_v[4 * j + 0, :]
            for k in range(1, A):
                lj = lj + pooled[k] * wb_v[4 * j + k, :]
            logits.append(lj + wb_v[A * A + j, :])
        m = _maxtree(logits)
        es = [jnp.exp(l - m) for l in logits]
        tot = (es[0] + es[1]) + (es[2] + es[3])
        rs = _rsqrt(tot)
        inv = rs * rs
        rhi4 = (lr0 >> 7) * 4
        cl = lr0 & 127
        for j in range(A):
            out_v[rhi4 + j, pl.ds(cl, LANES)] = es[j] * inv

    def blk_body(k, carry):
        lr0 = k * G2
        rhi = lr0 >> 7
        rlo = lr0 & 127
        acc_a = [jnp.zeros((LANES,), jnp.float32) for _ in range(P)]
        acc_b = [jnp.zeros((LANES,), jnp.float32) for _ in range(P)]
        n2a = jnp.zeros((LANES,), jnp.float32)
        n2b = jnp.zeros((LANES,), jnp.float32)
        for d in range(D):
            va = et_v[d >> 3, rhi, d & 7, pl.ds(rlo, LANES)]
            vb = et_v[d >> 3, rhi, d & 7, pl.ds(rlo + LANES, LANES)]
            for p in range(P):
                s = av_v[d * P + p, :]
                acc_a[p] = acc_a[p] + va * s
                acc_b[p] = acc_b[p] + vb * s
            n2a = n2a + va * va
            n2b = n2b + vb * vb
        finish(acc_a, n2a, lr0)
        finish(acc_b, n2b, lr0 + LANES)
        return carry

    lax.fori_loop(0, ROWS_W // G2, blk_body, 0)
    pltpu.sync_copy(out_v, out_hbm.at[pl.ds(wid * (ROWS_W // 128) * A, (ROWS_W // 128) * A)])


@functools.partial(
    pl.kernel,
    out_type=jax.ShapeDtypeStruct((B // 128 * A, 128), jnp.float32),
    mesh=plsc.VectorSubcoreMesh(core_axis_name="c", subcore_axis_name="s"),
    compiler_params=pltpu.CompilerParams(needs_layout_passes=False,
                                         use_tc_tiling_on_sc=False),
    scratch_types=[
        pltpu.VMEM((D // 8, ROWS_W // 128, 8, 128), jnp.float32),
        pltpu.VMEM((ROWS_W // 128 * A, 128), jnp.float32),
        pltpu.VMEM((D * P, LANES), jnp.float32),
        pltpu.VMEM((A * A + A, LANES), jnp.float32),
    ],
)
def _sc_decoder(et_hbm, av_hbm, wb_hbm, out_hbm, et_v, out_v, av_v, wb_v):
    _sc_body(et_hbm, av_hbm, wb_hbm, out_hbm, et_v, out_v, av_v, wb_v)


def kernel(embedded_words, action_vectors, W, b):
    av = action_vectors.reshape(D * P).astype(jnp.float32)
    av_splat = jnp.broadcast_to(av[:, None], (D * P, LANES))
    wb = jnp.concatenate([W.reshape(A * A), b.reshape(A)]).astype(jnp.float32)
    wb_splat = jnp.broadcast_to(wb[:, None], (A * A + A, LANES))
    et4 = embedded_words.reshape(128, 128, 8, 8).transpose(2, 0, 3, 1)
    o2 = _sc_decoder(et4, av_splat, wb_splat)
    return o2.reshape(B // 128, A, 128).transpose(0, 2, 1).reshape(B, A)


# speedup vs baseline: 11.5718x; 1.0533x over previous
import functools

import jax
import jax.numpy as jnp
from jax import lax
from jax.experimental import pallas as pl
from jax.experimental.pallas import tpu as pltpu
from jax.experimental.pallas import tpu_sc as plsc

B = 16384
D = 64
P = 11
A = 4
LANES = 16

NC = 2
NS = 16
NW = NC * NS
ROWS_W = B // NW
G2 = 32
EPS = 1e-8

GROUPS = ((0, 1, 2, 3), (4, 5, 6, 7, 8), (9,), (10,))


def _rsqrt(x):
    i = lax.bitcast_convert_type(x, jnp.int32)
    i = jnp.int32(0x5F3759DF) - (i >> 1)
    y = lax.bitcast_convert_type(i, jnp.float32)
    for _ in range(3):
        y = y * (1.5 - 0.5 * x * y * y)
    return y


def _maxtree(vals):
    vals = list(vals)
    while len(vals) > 1:
        vals = [jnp.maximum(vals[i], vals[i + 1]) for i in range(0, len(vals) - 1, 2)] + (
            [vals[-1]] if len(vals) % 2 else [])
    return vals[0]


def _sc_body(et_hbm, av_hbm, wb_hbm, out_hbm, et_v, out_v, av_v, wb_v,
             sem_e, sem_a):
    wid = lax.axis_index("s") * NC + lax.axis_index("c")

    cp_e = pltpu.async_copy(
        et_hbm.at[:, pl.ds(wid * (ROWS_W // 128), ROWS_W // 128)], et_v, sem_e)
    cp_a = pltpu.async_copy(av_hbm, av_v, sem_a)
    pltpu.sync_copy(wb_hbm, wb_v)
    cp_a.wait()

    inv_nb = []
    for p in range(P):
        nb2 = jnp.zeros((LANES,), jnp.float32)
        for d in range(D):
            s = av_v[d * P + p, :]
            nb2 = nb2 + s * s
        inv_nb.append(_rsqrt(jnp.maximum(nb2, EPS * EPS)))
    cp_e.wait()

    def finish(acc, n2, lr0):
        ca = _rsqrt(jnp.maximum(n2, EPS * EPS))
        sims = [acc[p] * inv_nb[p] for p in range(P)]
        pooled = [_maxtree([sims[p] for p in g]) * ca for g in GROUPS]
        logits = []
        for j in range(A):
            lj = pooled[0] * wb_v[4 * j + 0, :]
            for k in range(1, A):
                lj = lj + pooled[k] * wb_v[4 * j + k, :]
            logits.append(lj + wb_v[A * A + j, :])
        m = _maxtree(logits)
        es = [jnp.exp(l - m) for l in logits]
        tot = (es[0] + es[1]) + (es[2] + es[3])
        rs = _rsqrt(tot)
        inv = rs * rs
        rhi4 = (lr0 >> 7) * 4
        cl = lr0 & 127
        for j in range(A):
            out_v[rhi4 + j, pl.ds(cl, LANES)] = es[j] * inv

    def blk_body(k, carry):
        lr0 = k * G2
        rhi = lr0 >> 7
        rlo = lr0 & 127
        acc_a = [jnp.zeros((LANES,), jnp.float32) for _ in range(P)]
        acc_b = [jnp.zeros((LANES,), jnp.float32) for _ in range(P)]
        n2a = jnp.zeros((LANES,), jnp.float32)
        n2b = jnp.zeros((LANES,), jnp.float32)
        for d in range(D):
            va = et_v[d >> 3, rhi, d & 7, pl.ds(rlo, LANES)]
            vb = et_v[d >> 3, rhi, d & 7, pl.ds(rlo + LANES, LANES)]
            for p in range(P):
                s = av_v[d * P + p, :]
                acc_a[p] = acc_a[p] + va * s
                acc_b[p] = acc_b[p] + vb * s
            n2a = n2a + va * va
            n2b = n2b + vb * vb
        finish(acc_a, n2a, lr0)
        finish(acc_b, n2b, lr0 + LANES)
        return carry

    lax.fori_loop(0, ROWS_W // G2, blk_body, 0)
    pltpu.sync_copy(out_v, out_hbm.at[pl.ds(wid * (ROWS_W // 128) * A, (ROWS_W // 128) * A)])


@functools.partial(
    pl.kernel,
    out_type=jax.ShapeDtypeStruct((B // 128 * A, 128), jnp.float32),
    mesh=plsc.VectorSubcoreMesh(core_axis_name="c", subcore_axis_name="s"),
    compiler_params=pltpu.CompilerParams(needs_layout_passes=False,
                                         use_tc_tiling_on_sc=False),
    scratch_types=[
        pltpu.VMEM((D // 8, ROWS_W // 128, 8, 128), jnp.float32),
        pltpu.VMEM((ROWS_W // 128 * A, 128), jnp.float32),
        pltpu.VMEM((D * P, LANES), jnp.float32),
        pltpu.VMEM((A * A + A, LANES), jnp.float32),
        pltpu.SemaphoreType.DMA,
        pltpu.SemaphoreType.DMA,
    ],
)
def _sc_decoder(et_hbm, av_hbm, wb_hbm, out_hbm, et_v, out_v, av_v, wb_v,
                sem_e, sem_a):
    _sc_body(et_hbm, av_hbm, wb_hbm, out_hbm, et_v, out_v, av_v, wb_v,
             sem_e, sem_a)


def kernel(embedded_words, action_vectors, W, b):
    av = action_vectors.reshape(D * P).astype(jnp.float32)
    av_splat = jnp.broadcast_to(av[:, None], (D * P, LANES))
    wb = jnp.concatenate([W.reshape(A * A), b.reshape(A)]).astype(jnp.float32)
    wb_splat = jnp.broadcast_to(wb[:, None], (A * A + A, LANES))
    et4 = embedded_words.reshape(128, 128, 8, 8).transpose(2, 0, 3, 1)
    o2 = _sc_decoder(et4, av_splat, wb_splat)
    return o2.reshape(B // 128, A, 128).transpose(0, 2, 1).reshape(B, A)
